# baseline (speedup 1.0000x reference)
import functools

import jax
import jax.numpy as jnp
from jax import lax
from jax.experimental import pallas as pl
from jax.experimental.pallas import tpu as pltpu
from jax.experimental.pallas import tpu_sc as plsc

E = 16
T = 2048
D = 1024
H = 1024
K = 2
N = T * K
BT = 256
NB = N // BT
MAXT = NB + E - 1

_NC = 2
_NS = 16
_NW = _NC * _NS


def _routing_metadata(weights, indices):
    eid = indices.reshape(-1).astype(jnp.int32)
    oh = (eid[:, None] == jnp.arange(E, dtype=jnp.int32)[None, :]).astype(
        jnp.int32)
    ranks = jnp.cumsum(oh, axis=0)
    rank = jnp.sum(jnp.where(oh != 0, ranks, 0), axis=1) - 1
    offs = jnp.sum(
        eid[None, :] < jnp.arange(E + 1, dtype=jnp.int32)[:, None],
        axis=1, dtype=jnp.int32)
    inv = (offs[eid] + rank).astype(jnp.int32)
    inv2 = inv.reshape(T, K).T.reshape(-1)
    counts = offs[1:] - offs[:E]
    fb = offs[:E] // BT
    lb = jnp.maximum(offs[1:] - 1, 0) // BT
    tiles_per = jnp.where(counts > 0, lb - fb + 1, 0).astype(jnp.int32)
    toffs = jnp.concatenate(
        [jnp.zeros((1,), jnp.int32), jnp.cumsum(tiles_per).astype(jnp.int32)])
    total = toffs[E]
    ti = jnp.arange(MAXT, dtype=jnp.int32)
    e_raw = jnp.clip(
        jnp.searchsorted(toffs, ti, side="right",
                         method="compare_all").astype(jnp.int32) - 1,
        0, E - 1)
    b_raw = fb[e_raw] + (ti - toffs[e_raw])
    valid = ti < total
    e_last = jnp.max(jnp.where(counts > 0, jnp.arange(E, dtype=jnp.int32), -1))
    eid_t = jnp.where(valid, e_raw, e_last).astype(jnp.int32)
    blk_t = jnp.where(valid, b_raw, NB - 1).astype(jnp.int32)
    lo_g = jnp.maximum(offs[eid_t], blk_t * BT)
    hi_g = jnp.minimum(offs[eid_t + 1], (blk_t + 1) * BT)
    lo_t = jnp.where(valid, lo_g - blk_t * BT, 0).astype(jnp.int32)
    hi_t = jnp.where(valid, hi_g - blk_t * BT, 0).astype(jnp.int32)
    first_t = jnp.concatenate(
        [jnp.ones((1,), jnp.int32),
         (blk_t[1:] != blk_t[:-1]).astype(jnp.int32)])
    run_start = jnp.concatenate(
        [jnp.ones((1,), jnp.int32),
         (eid_t[1:] != eid_t[:-1]).astype(jnp.int32)])
    run_id = jnp.cumsum(run_start).astype(jnp.int32) - 1
    nruns = run_id[MAXT - 1] + 1
    rex = jnp.zeros((MAXT,), jnp.int32).at[run_id].set(eid_t)
    nr = run_id + 1
    wait_t = run_start
    slot_t = run_id % 2
    pref_t = (run_start * (nr < nruns)).astype(jnp.int32)
    prefe_t = rex[jnp.minimum(nr, MAXT - 1)]
    prefs_t = nr % 2
    return (inv2, blk_t, eid_t, lo_t, hi_t, first_t,
            wait_t, slot_t, pref_t, prefe_t, prefs_t)


def _sc_dispatch(x, inv2):
    tpw = T // _NW
    ct = 16
    nch = tpw // ct
    mesh = plsc.VectorSubcoreMesh(core_axis_name="c", subcore_axis_name="s")

    @functools.partial(
        pl.kernel,
        mesh=mesh,
        out_type=jax.ShapeDtypeStruct((N, D), jnp.float32),
        scratch_types=[
            pltpu.VMEM((ct, D), jnp.float32),
            pltpu.VMEM((ct, D), jnp.float32),
            pltpu.VMEM((ct,), jnp.int32),
            pltpu.VMEM((ct,), jnp.int32),
            pltpu.VMEM((ct,), jnp.int32),
            pltpu.VMEM((ct,), jnp.int32),
            pltpu.SemaphoreType.DMA,
            pltpu.SemaphoreType.DMA,
            pltpu.SemaphoreType.DMA,
            pltpu.SemaphoreType.DMA,
            pltpu.SemaphoreType.DMA,
            pltpu.SemaphoreType.DMA,
        ],
    )
    def k(x_hbm, inv2_hbm, xs_hbm, xb0, xb1, ie0, ie1, io0, io1,
          sx0, sx1, se0, se1, so0, so1):
        wid = lax.axis_index("s") * _NC + lax.axis_index("c")
        tbase = wid * tpw
        xbufs = (xb0, xb1)
        iebs = (ie0, ie1)
        iobs = (io0, io1)
        sxs = (sx0, sx1)
        ses = (se0, se1)
        sos = (so0, so1)
        loads = [None, None]
        sc_e = [None, None]
        sc_o = [None, None]
        loads[0] = pltpu.async_copy(
            x_hbm.at[pl.ds(tbase, ct)], xb0, sx0)
        for c in range(nch):
            s = c % 2
            off = tbase + c * ct
            pltpu.sync_copy(inv2_hbm.at[pl.ds(off, ct)], iebs[s])
            pltpu.sync_copy(inv2_hbm.at[pl.ds(T + off, ct)], iobs[s])
            loads[s].wait()
            if c + 1 < nch:
                s1 = (c + 1) % 2
                if sc_e[s1] is not None:
                    sc_e[s1].wait()
                    sc_o[s1].wait()
                loads[s1] = pltpu.async_copy(
                    x_hbm.at[pl.ds(tbase + (c + 1) * ct, ct)],
                    xbufs[s1], sxs[s1])
            sc_e[s] = pltpu.async_copy(xbufs[s], xs_hbm.at[iebs[s]], ses[s])
            sc_o[s] = pltpu.async_copy(xbufs[s], xs_hbm.at[iobs[s]], sos[s])
        for s in range(2):
            if sc_e[s] is not None:
                sc_e[s].wait()
                sc_o[s].wait()

    return k(x, inv2)


def _sc_pair_add(ysw, inv2, weights):
    tpw = T // _NW
    ct = 16
    nch = tpw // ct
    mesh = plsc.VectorSubcoreMesh(core_axis_name="c", subcore_axis_name="s")

    @functools.partial(
        pl.kernel,
        mesh=mesh,
        out_type=jax.ShapeDtypeStruct((T, D), jnp.float32),
        scratch_types=[
            pltpu.VMEM((2 * ct,), jnp.int32),
            pltpu.VMEM((2 * ct,), jnp.int32),
            pltpu.VMEM((tpw * K + 16,), jnp.float32),
            pltpu.VMEM((2 * ct, D), jnp.float32),
            pltpu.VMEM((2 * ct, D), jnp.float32),
            pltpu.VMEM((ct, D), jnp.float32),
            pltpu.VMEM((ct, D), jnp.float32),
            pltpu.SemaphoreType.DMA,
            pltpu.SemaphoreType.DMA,
            pltpu.SemaphoreType.DMA,
            pltpu.SemaphoreType.DMA,
        ],
    )
    def k(ysw_hbm, inv2_hbm, w_hbm, y_hbm, ix0, ix1, wv, buf0, buf1, ob0, ob1,
          sg0, sg1, sw0, sw1):
        wid = lax.axis_index("s") * _NC + lax.axis_index("c")
        tbase = wid * tpw
        ixs = (ix0, ix1)
        bufs = (buf0, buf1)
        obs = (ob0, ob1)
        sgs = (sg0, sg1)
        sws = (sw0, sw1)
        pltpu.sync_copy(w_hbm.at[pl.ds(tbase * K, tpw * K)],
                        wv.at[pl.ds(0, tpw * K)])
        gathers = [None, None]
        writes = [None, None]
        pltpu.sync_copy(inv2_hbm.at[pl.ds(tbase, ct)], ix0.at[pl.ds(0, ct)])
        pltpu.sync_copy(inv2_hbm.at[pl.ds(T + tbase, ct)],
                        ix0.at[pl.ds(ct, ct)])
        gathers[0] = pltpu.async_copy(ysw_hbm.at[ix0], buf0, sg0)
        for c in range(nch):
            s = c % 2
            gathers[s].wait()
            if c + 1 < nch:
                s1 = (c + 1) % 2
                off1 = tbase + (c + 1) * ct
                pltpu.sync_copy(inv2_hbm.at[pl.ds(off1, ct)],
                                ixs[s1].at[pl.ds(0, ct)])
                pltpu.sync_copy(inv2_hbm.at[pl.ds(T + off1, ct)],
                                ixs[s1].at[pl.ds(ct, ct)])
                gathers[s1] = pltpu.async_copy(
                    ysw_hbm.at[ixs[s1]], bufs[s1], sgs[s1])
            if writes[s] is not None:
                writes[s].wait()
            buf = bufs[s]
            ob = obs[s]
            wbase = c * ct

            def body(j, carry, buf=buf, ob=ob, wbase=wbase):
                wvec = wv[pl.ds(K * (wbase + j), 16)]
                w0 = wvec[0]
                w1 = wvec[1]
                for u in range(D // 16):
                    v = u * 16
                    ob[j, pl.ds(v, 16)] = (
                        w0 * buf[j, pl.ds(v, 16)]
                        + w1 * buf[ct + j, pl.ds(v, 16)])
                return carry

            lax.fori_loop(0, ct, body, 0)
            writes[s] = pltpu.async_copy(
                ob, y_hbm.at[pl.ds(tbase + c * ct, ct)], sws[s])
        writes[0].wait()
        writes[1].wait()

    return k(ysw, inv2, weights.reshape(-1))


def _gmm_body(blk_r, eid_r, lo_r, hi_r, first_r,
              wait_r, slot_r, pref_r, prefe_r, prefs_r,
              xs_r, w1_hbm, w2_hbm, out_r,
              w1b, w2b, sem1, sem2):
    i = pl.program_id(0)

    @pl.when(i == 0)
    def _():
        e0 = eid_r[0]
        pltpu.make_async_copy(w1_hbm.at[e0], w1b.at[0], sem1.at[0]).start()
        pltpu.make_async_copy(w2_hbm.at[e0], w2b.at[0], sem2.at[0]).start()

    @pl.when(pref_r[i] != 0)
    def _():
        e = prefe_r[i]
        s = prefs_r[i]
        pltpu.make_async_copy(w1_hbm.at[e], w1b.at[s], sem1.at[s]).start()
        pltpu.make_async_copy(w2_hbm.at[e], w2b.at[s], sem2.at[s]).start()

    @pl.when(wait_r[i] != 0)
    def _():
        s = slot_r[i]
        pltpu.make_async_copy(w1_hbm.at[0], w1b.at[s], sem1.at[s]).wait()
        pltpu.make_async_copy(w2_hbm.at[0], w2b.at[s], sem2.at[s]).wait()

    cs = slot_r[i]
    lo = lo_r[i]
    hi = hi_r[i]
    rows = lax.broadcasted_iota(jnp.int32, (BT, 1), 0)
    mask = (rows >= lo) & (rows < hi)
    xb = jnp.where(mask, xs_r[...], 0.0)
    h = jnp.dot(xb, w1b[cs], preferred_element_type=jnp.float32)
    yv = h[:, :H]
    g = h[:, H:]
    act = yv * (g * jax.nn.sigmoid(g))
    o = jnp.dot(act, w2b[cs], preferred_element_type=jnp.float32)

    @pl.when(first_r[i] != 0)
    def _():
        out_r[...] = o

    @pl.when(first_r[i] == 0)
    def _():
        out_r[...] = out_r[...] + o


def _grouped_mlp(xs, W1, W2, blk_t, eid_t, lo_t, hi_t, first_t,
                 wait_t, slot_t, pref_t, prefe_t, prefs_t):
    grid_spec = pltpu.PrefetchScalarGridSpec(
        num_scalar_prefetch=10,
        grid=(MAXT,),
        in_specs=[
            pl.BlockSpec((BT, D), lambda i, *s: (s[0][i], 0)),
            pl.BlockSpec(memory_space=pl.ANY),
            pl.BlockSpec(memory_space=pl.ANY),
        ],
        out_specs=pl.BlockSpec((BT, D), lambda i, *s: (s[0][i], 0)),
        scratch_shapes=[
            pltpu.VMEM((2, D, 2 * H), jnp.float32),
            pltpu.VMEM((2, H, D), jnp.float32),
            pltpu.SemaphoreType.DMA((2,)),
            pltpu.SemaphoreType.DMA((2,)),
        ],
    )
    return pl.pallas_call(
        _gmm_body,
        grid_spec=grid_spec,
        out_shape=jax.ShapeDtypeStruct((N, D), jnp.float32),
    )(blk_t, eid_t, lo_t, hi_t, first_t,
      wait_t, slot_t, pref_t, prefe_t, prefs_t,
      xs, W1, W2)


def kernel(x, weights, indices, W1, W2):
    (inv2, blk_t, eid_t, lo_t, hi_t, first_t,
     wait_t, slot_t, pref_t, prefe_t, prefs_t) = _routing_metadata(
        weights, indices)
    xs = _sc_dispatch(x, inv2)
    ys = _grouped_mlp(xs, W1, W2, blk_t, eid_t, lo_t, hi_t, first_t,
                      wait_t, slot_t, pref_t, prefe_t, prefs_t)
    return _sc_pair_add(ys, inv2, weights)


# speedup vs baseline: 1.0007x; 1.0007x over previous
import functools

import jax
import jax.numpy as jnp
from jax import lax
from jax.experimental import pallas as pl
from jax.experimental.pallas import tpu as pltpu
from jax.experimental.pallas import tpu_sc as plsc

E = 16
T = 2048
D = 1024
H = 1024
K = 2
N = T * K
BT = 256
NB = N // BT
MAXT = NB + E - 1

_NC = 2
_NS = 16
_NW = _NC * _NS


def _routing_metadata(weights, indices):
    eid = indices.reshape(-1).astype(jnp.int32)
    oh = (eid[:, None] == jnp.arange(E, dtype=jnp.int32)[None, :]).astype(
        jnp.int32)
    ranks = jnp.cumsum(oh, axis=0)
    rank = jnp.sum(jnp.where(oh != 0, ranks, 0), axis=1) - 1
    offs = jnp.sum(
        eid[None, :] < jnp.arange(E + 1, dtype=jnp.int32)[:, None],
        axis=1, dtype=jnp.int32)
    inv = (offs[eid] + rank).astype(jnp.int32)
    inv2 = inv.reshape(T, K).T.reshape(-1)
    counts = offs[1:] - offs[:E]
    fb = offs[:E] // BT
    lb = jnp.maximum(offs[1:] - 1, 0) // BT
    tiles_per = jnp.where(counts > 0, lb - fb + 1, 0).astype(jnp.int32)
    toffs = jnp.concatenate(
        [jnp.zeros((1,), jnp.int32), jnp.cumsum(tiles_per).astype(jnp.int32)])
    total = toffs[E]
    ti = jnp.arange(MAXT, dtype=jnp.int32)
    e_raw = jnp.clip(
        jnp.searchsorted(toffs, ti, side="right",
                         method="compare_all").astype(jnp.int32) - 1,
        0, E - 1)
    b_raw = fb[e_raw] + (ti - toffs[e_raw])
    valid = ti < total
    e_last = jnp.max(jnp.where(counts > 0, jnp.arange(E, dtype=jnp.int32), -1))
    eid_t = jnp.where(valid, e_raw, e_last).astype(jnp.int32)
    blk_t = jnp.where(valid, b_raw, NB - 1).astype(jnp.int32)
    lo_g = jnp.maximum(offs[eid_t], blk_t * BT)
    hi_g = jnp.minimum(offs[eid_t + 1], (blk_t + 1) * BT)
    lo_t = jnp.where(valid, lo_g - blk_t * BT, 0).astype(jnp.int32)
    hi_t = jnp.where(valid, hi_g - blk_t * BT, 0).astype(jnp.int32)
    first_t = jnp.concatenate(
        [jnp.ones((1,), jnp.int32),
         (blk_t[1:] != blk_t[:-1]).astype(jnp.int32)])
    run_start = jnp.concatenate(
        [jnp.ones((1,), jnp.int32),
         (eid_t[1:] != eid_t[:-1]).astype(jnp.int32)])
    run_id = jnp.cumsum(run_start).astype(jnp.int32) - 1
    nruns = run_id[MAXT - 1] + 1
    rex = jnp.zeros((MAXT,), jnp.int32).at[run_id].set(eid_t)
    nr = run_id + 1
    wait_t = run_start
    slot_t = run_id % 2
    pref_t = (run_start * (nr < nruns)).astype(jnp.int32)
    prefe_t = rex[jnp.minimum(nr, MAXT - 1)]
    prefs_t = nr % 2
    return (inv2, blk_t, eid_t, lo_t, hi_t, first_t,
            wait_t, slot_t, pref_t, prefe_t, prefs_t)


def _sc_dispatch(x, inv2):
    tpw = T // _NW
    ct = 16
    nch = tpw // ct
    mesh = plsc.VectorSubcoreMesh(core_axis_name="c", subcore_axis_name="s")

    @functools.partial(
        pl.kernel,
        mesh=mesh,
        out_type=jax.ShapeDtypeStruct((N, D), jnp.float32),
        scratch_types=[
            pltpu.VMEM((ct, D), jnp.float32),
            pltpu.VMEM((ct, D), jnp.float32),
            pltpu.VMEM((ct,), jnp.int32),
            pltpu.VMEM((ct,), jnp.int32),
            pltpu.VMEM((ct,), jnp.int32),
            pltpu.VMEM((ct,), jnp.int32),
            pltpu.SemaphoreType.DMA,
            pltpu.SemaphoreType.DMA,
            pltpu.SemaphoreType.DMA,
            pltpu.SemaphoreType.DMA,
            pltpu.SemaphoreType.DMA,
            pltpu.SemaphoreType.DMA,
        ],
    )
    def k(x_hbm, inv2_hbm, xs_hbm, xb0, xb1, ie0, ie1, io0, io1,
          sx0, sx1, se0, se1, so0, so1):
        wid = lax.axis_index("s") * _NC + lax.axis_index("c")
        tbase = wid * tpw
        xbufs = (xb0, xb1)
        iebs = (ie0, ie1)
        iobs = (io0, io1)
        sxs = (sx0, sx1)
        ses = (se0, se1)
        sos = (so0, so1)
        loads = [None, None]
        sc_e = [None, None]
        sc_o = [None, None]
        loads[0] = pltpu.async_copy(
            x_hbm.at[pl.ds(tbase, ct)], xb0, sx0)
        for c in range(nch):
            s = c % 2
            off = tbase + c * ct
            pltpu.sync_copy(inv2_hbm.at[pl.ds(off, ct)], iebs[s])
            pltpu.sync_copy(inv2_hbm.at[pl.ds(T + off, ct)], iobs[s])
            loads[s].wait()
            if c + 1 < nch:
                s1 = (c + 1) % 2
                if sc_e[s1] is not None:
                    sc_e[s1].wait()
                    sc_o[s1].wait()
                loads[s1] = pltpu.async_copy(
                    x_hbm.at[pl.ds(tbase + (c + 1) * ct, ct)],
                    xbufs[s1], sxs[s1])
            sc_e[s] = pltpu.async_copy(xbufs[s], xs_hbm.at[iebs[s]], ses[s])
            sc_o[s] = pltpu.async_copy(xbufs[s], xs_hbm.at[iobs[s]], sos[s])
        for s in range(2):
            if sc_e[s] is not None:
                sc_e[s].wait()
                sc_o[s].wait()

    return k(x, inv2)


def _sc_pair_add(ysw, inv2, weights):
    tpw = T // _NW
    ct = 16
    nch = tpw // ct
    mesh = plsc.VectorSubcoreMesh(core_axis_name="c", subcore_axis_name="s")

    @functools.partial(
        pl.kernel,
        mesh=mesh,
        out_type=jax.ShapeDtypeStruct((T, D), jnp.float32),
        scratch_types=[
            pltpu.VMEM((2 * ct,), jnp.int32),
            pltpu.VMEM((2 * ct,), jnp.int32),
            pltpu.VMEM((tpw * K + 16,), jnp.float32),
            pltpu.VMEM((2 * ct, D), jnp.float32),
            pltpu.VMEM((2 * ct, D), jnp.float32),
            pltpu.VMEM((ct, D), jnp.float32),
            pltpu.VMEM((ct, D), jnp.float32),
            pltpu.SemaphoreType.DMA,
            pltpu.SemaphoreType.DMA,
            pltpu.SemaphoreType.DMA,
            pltpu.SemaphoreType.DMA,
        ],
    )
    def k(ysw_hbm, inv2_hbm, w_hbm, y_hbm, ix0, ix1, wv, buf0, buf1, ob0, ob1,
          sg0, sg1, sw0, sw1):
        wid = lax.axis_index("s") * _NC + lax.axis_index("c")
        tbase = wid * tpw
        ixs = (ix0, ix1)
        bufs = (buf0, buf1)
        obs = (ob0, ob1)
        sgs = (sg0, sg1)
        sws = (sw0, sw1)
        pltpu.sync_copy(w_hbm.at[pl.ds(tbase * K, tpw * K)],
                        wv.at[pl.ds(0, tpw * K)])
        gathers = [None, None]
        writes = [None, None]
        pltpu.sync_copy(inv2_hbm.at[pl.ds(tbase, ct)], ix0.at[pl.ds(0, ct)])
        pltpu.sync_copy(inv2_hbm.at[pl.ds(T + tbase, ct)],
                        ix0.at[pl.ds(ct, ct)])
        gathers[0] = pltpu.async_copy(ysw_hbm.at[ix0], buf0, sg0)
        for c in range(nch):
            s = c % 2
            gathers[s].wait()
            if c + 1 < nch:
                s1 = (c + 1) % 2
                off1 = tbase + (c + 1) * ct
                pltpu.sync_copy(inv2_hbm.at[pl.ds(off1, ct)],
                                ixs[s1].at[pl.ds(0, ct)])
                pltpu.sync_copy(inv2_hbm.at[pl.ds(T + off1, ct)],
                                ixs[s1].at[pl.ds(ct, ct)])
                gathers[s1] = pltpu.async_copy(
                    ysw_hbm.at[ixs[s1]], bufs[s1], sgs[s1])
            if writes[s] is not None:
                writes[s].wait()
            buf = bufs[s]
            ob = obs[s]
            wbase = c * ct

            def body(j, carry, buf=buf, ob=ob, wbase=wbase):
                wvec = wv[pl.ds(K * (wbase + j), 16)]
                w0 = wvec[0]
                w1 = wvec[1]
                for u in range(D // 16):
                    v = u * 16
                    ob[j, pl.ds(v, 16)] = (
                        w0 * buf[j, pl.ds(v, 16)]
                        + w1 * buf[ct + j, pl.ds(v, 16)])
                return carry

            lax.fori_loop(0, ct, body, 0)
            writes[s] = pltpu.async_copy(
                ob, y_hbm.at[pl.ds(tbase + c * ct, ct)], sws[s])
        writes[0].wait()
        writes[1].wait()

    return k(ysw, inv2, weights.reshape(-1))


def _gmm_body(blk_r, eid_r, lo_r, hi_r, first_r,
              wait_r, slot_r, pref_r, prefe_r, prefs_r,
              xs_r, w1_hbm, w2_hbm, out_r,
              w1b, w2b, sem1, sem2):
    i = pl.program_id(0)
    hd = D // 2

    def _w_start(e, s):
        pltpu.make_async_copy(w1_hbm.at[e, pl.ds(0, hd)],
                              w1b.at[s, pl.ds(0, hd)], sem1.at[s]).start()
        pltpu.make_async_copy(w1_hbm.at[e, pl.ds(hd, hd)],
                              w1b.at[s, pl.ds(hd, hd)], sem1.at[s]).start()
        pltpu.make_async_copy(w2_hbm.at[e, pl.ds(0, hd)],
                              w2b.at[s, pl.ds(0, hd)], sem2.at[s]).start()
        pltpu.make_async_copy(w2_hbm.at[e, pl.ds(hd, hd)],
                              w2b.at[s, pl.ds(hd, hd)], sem2.at[s]).start()

    @pl.when(i == 0)
    def _():
        _w_start(eid_r[0], 0)

    @pl.when(pref_r[i] != 0)
    def _():
        _w_start(prefe_r[i], prefs_r[i])

    @pl.when(wait_r[i] != 0)
    def _():
        s = slot_r[i]
        pltpu.make_async_copy(w1_hbm.at[0, pl.ds(0, hd)],
                              w1b.at[s, pl.ds(0, hd)], sem1.at[s]).wait()
        pltpu.make_async_copy(w1_hbm.at[0, pl.ds(hd, hd)],
                              w1b.at[s, pl.ds(hd, hd)], sem1.at[s]).wait()
        pltpu.make_async_copy(w2_hbm.at[0, pl.ds(0, hd)],
                              w2b.at[s, pl.ds(0, hd)], sem2.at[s]).wait()
        pltpu.make_async_copy(w2_hbm.at[0, pl.ds(hd, hd)],
                              w2b.at[s, pl.ds(hd, hd)], sem2.at[s]).wait()

    cs = slot_r[i]
    lo = lo_r[i]
    hi = hi_r[i]
    rows = lax.broadcasted_iota(jnp.int32, (BT, 1), 0)
    mask = (rows >= lo) & (rows < hi)
    xb = jnp.where(mask, xs_r[...], 0.0)
    h = jnp.dot(xb, w1b[cs], preferred_element_type=jnp.float32)
    yv = h[:, :H]
    g = h[:, H:]
    act = yv * (g * jax.nn.sigmoid(g))
    o = jnp.dot(act, w2b[cs], preferred_element_type=jnp.float32)

    @pl.when(first_r[i] != 0)
    def _():
        out_r[...] = o

    @pl.when(first_r[i] == 0)
    def _():
        out_r[...] = out_r[...] + o


def _grouped_mlp(xs, W1, W2, blk_t, eid_t, lo_t, hi_t, first_t,
                 wait_t, slot_t, pref_t, prefe_t, prefs_t):
    grid_spec = pltpu.PrefetchScalarGridSpec(
        num_scalar_prefetch=10,
        grid=(MAXT,),
        in_specs=[
            pl.BlockSpec((BT, D), lambda i, *s: (s[0][i], 0)),
            pl.BlockSpec(memory_space=pl.ANY),
            pl.BlockSpec(memory_space=pl.ANY),
        ],
        out_specs=pl.BlockSpec((BT, D), lambda i, *s: (s[0][i], 0)),
        scratch_shapes=[
            pltpu.VMEM((2, D, 2 * H), jnp.float32),
            pltpu.VMEM((2, H, D), jnp.float32),
            pltpu.SemaphoreType.DMA((2,)),
            pltpu.SemaphoreType.DMA((2,)),
        ],
    )
    return pl.pallas_call(
        _gmm_body,
        grid_spec=grid_spec,
        out_shape=jax.ShapeDtypeStruct((N, D), jnp.float32),
    )(blk_t, eid_t, lo_t, hi_t, first_t,
      wait_t, slot_t, pref_t, prefe_t, prefs_t,
      xs, W1, W2)


def kernel(x, weights, indices, W1, W2):
    (inv2, blk_t, eid_t, lo_t, hi_t, first_t,
     wait_t, slot_t, pref_t, prefe_t, prefs_t) = _routing_metadata(
        weights, indices)
    xs = _sc_dispatch(x, inv2)
    ys = _grouped_mlp(xs, W1, W2, blk_t, eid_t, lo_t, hi_t, first_t,
                      wait_t, slot_t, pref_t, prefe_t, prefs_t)
    return _sc_pair_add(ys, inv2, weights)


# speedup vs baseline: 1.1239x; 1.1232x over previous
import functools

import jax
import jax.numpy as jnp
from jax import lax
from jax.experimental import pallas as pl
from jax.experimental.pallas import tpu as pltpu
from jax.experimental.pallas import tpu_sc as plsc

E = 16
T = 2048
D = 1024
H = 1024
K = 2
N = T * K
BT = 256
NB = N // BT
MAXT = NB + E - 1

_NC = 2
_NS = 16
_NW = _NC * _NS


def _routing_metadata(weights, indices):
    eid = indices.reshape(-1).astype(jnp.int32)
    oh = (eid[:, None] == jnp.arange(E, dtype=jnp.int32)[None, :]).astype(
        jnp.int32)
    ranks = jnp.cumsum(oh, axis=0)
    offs = jnp.sum(
        eid[None, :] < jnp.arange(E + 1, dtype=jnp.int32)[:, None],
        axis=1, dtype=jnp.int32)
    inv = jnp.sum(
        jnp.where(oh != 0, ranks + offs[None, :E] - 1, 0),
        axis=1).astype(jnp.int32)
    inv2 = inv.reshape(T, K).T.reshape(-1)
    counts = offs[1:] - offs[:E]
    fb = offs[:E] // BT
    lb = jnp.maximum(offs[1:] - 1, 0) // BT
    tiles_per = jnp.where(counts > 0, lb - fb + 1, 0).astype(jnp.int32)
    toffs = jnp.concatenate(
        [jnp.zeros((1,), jnp.int32), jnp.cumsum(tiles_per).astype(jnp.int32)])
    total = toffs[E]
    ti = jnp.arange(MAXT, dtype=jnp.int32)
    e_raw = jnp.clip(
        jnp.searchsorted(toffs, ti, side="right",
                         method="compare_all").astype(jnp.int32) - 1,
        0, E - 1)
    b_raw = fb[e_raw] + (ti - toffs[e_raw])
    valid = ti < total
    e_last = jnp.max(jnp.where(counts > 0, jnp.arange(E, dtype=jnp.int32), -1))
    eid_t = jnp.where(valid, e_raw, e_last).astype(jnp.int32)
    blk_t = jnp.where(valid, b_raw, NB - 1).astype(jnp.int32)
    lo_g = jnp.maximum(offs[eid_t], blk_t * BT)
    hi_g = jnp.minimum(offs[eid_t + 1], (blk_t + 1) * BT)
    lo_t = jnp.where(valid, lo_g - blk_t * BT, 0).astype(jnp.int32)
    hi_t = jnp.where(valid, hi_g - blk_t * BT, 0).astype(jnp.int32)
    first_t = jnp.concatenate(
        [jnp.ones((1,), jnp.int32),
         (blk_t[1:] != blk_t[:-1]).astype(jnp.int32)])
    run_start = jnp.concatenate(
        [jnp.ones((1,), jnp.int32),
         (eid_t[1:] != eid_t[:-1]).astype(jnp.int32)])
    run_id = jnp.cumsum(run_start).astype(jnp.int32) - 1
    nruns = run_id[MAXT - 1] + 1
    rex = jnp.zeros((MAXT,), jnp.int32).at[run_id].set(eid_t)
    nr = run_id + 1
    wait_t = run_start
    slot_t = run_id % 2
    pref_t = (run_start * (nr < nruns)).astype(jnp.int32)
    prefe_t = rex[jnp.minimum(nr, MAXT - 1)]
    prefs_t = nr % 2
    return (inv2, blk_t, eid_t, lo_t, hi_t, first_t,
            wait_t, slot_t, pref_t, prefe_t, prefs_t)


def _sc_dispatch(x, inv2):
    tpw = T // _NW
    ct = 16
    nch = tpw // ct
    mesh = plsc.VectorSubcoreMesh(core_axis_name="c", subcore_axis_name="s")

    @functools.partial(
        pl.kernel,
        mesh=mesh,
        out_type=jax.ShapeDtypeStruct((N, D), jnp.float32),
        scratch_types=[
            pltpu.VMEM((ct, D), jnp.float32),
            pltpu.VMEM((ct, D), jnp.float32),
            pltpu.VMEM((ct,), jnp.int32),
            pltpu.VMEM((ct,), jnp.int32),
            pltpu.VMEM((ct,), jnp.int32),
            pltpu.VMEM((ct,), jnp.int32),
            pltpu.SemaphoreType.DMA,
            pltpu.SemaphoreType.DMA,
            pltpu.SemaphoreType.DMA,
            pltpu.SemaphoreType.DMA,
            pltpu.SemaphoreType.DMA,
            pltpu.SemaphoreType.DMA,
        ],
    )
    def k(x_hbm, inv2_hbm, xs_hbm, xb0, xb1, ie0, ie1, io0, io1,
          sx0, sx1, se0, se1, so0, so1):
        wid = lax.axis_index("s") * _NC + lax.axis_index("c")
        tbase = wid * tpw
        xbufs = (xb0, xb1)
        iebs = (ie0, ie1)
        iobs = (io0, io1)
        sxs = (sx0, sx1)
        ses = (se0, se1)
        sos = (so0, so1)
        loads = [None, None]
        sc_e = [None, None]
        sc_o = [None, None]
        loads[0] = pltpu.async_copy(
            x_hbm.at[pl.ds(tbase, ct)], xb0, sx0)
        for c in range(nch):
            s = c % 2
            off = tbase + c * ct
            pltpu.sync_copy(inv2_hbm.at[pl.ds(off, ct)], iebs[s])
            pltpu.sync_copy(inv2_hbm.at[pl.ds(T + off, ct)], iobs[s])
            loads[s].wait()
            if c + 1 < nch:
                s1 = (c + 1) % 2
                if sc_e[s1] is not None:
                    sc_e[s1].wait()
                    sc_o[s1].wait()
                loads[s1] = pltpu.async_copy(
                    x_hbm.at[pl.ds(tbase + (c + 1) * ct, ct)],
                    xbufs[s1], sxs[s1])
            sc_e[s] = pltpu.async_copy(xbufs[s], xs_hbm.at[iebs[s]], ses[s])
            sc_o[s] = pltpu.async_copy(xbufs[s], xs_hbm.at[iobs[s]], sos[s])
        for s in range(2):
            if sc_e[s] is not None:
                sc_e[s].wait()
                sc_o[s].wait()

    return k(x, inv2)


def _sc_pair_add(ysw, inv2, weights):
    tpw = T // _NW
    ct = 16
    nch = tpw // ct
    mesh = plsc.VectorSubcoreMesh(core_axis_name="c", subcore_axis_name="s")

    @functools.partial(
        pl.kernel,
        mesh=mesh,
        out_type=jax.ShapeDtypeStruct((T, D), jnp.float32),
        scratch_types=[
            pltpu.VMEM((2 * ct,), jnp.int32),
            pltpu.VMEM((2 * ct,), jnp.int32),
            pltpu.VMEM((tpw * K + 16,), jnp.float32),
            pltpu.VMEM((2 * ct, D), jnp.float32),
            pltpu.VMEM((2 * ct, D), jnp.float32),
            pltpu.VMEM((ct, D), jnp.float32),
            pltpu.VMEM((ct, D), jnp.float32),
            pltpu.SemaphoreType.DMA,
            pltpu.SemaphoreType.DMA,
            pltpu.SemaphoreType.DMA,
            pltpu.SemaphoreType.DMA,
        ],
    )
    def k(ysw_hbm, inv2_hbm, w_hbm, y_hbm, ix0, ix1, wv, buf0, buf1, ob0, ob1,
          sg0, sg1, sw0, sw1):
        wid = lax.axis_index("s") * _NC + lax.axis_index("c")
        tbase = wid * tpw
        ixs = (ix0, ix1)
        bufs = (buf0, buf1)
        obs = (ob0, ob1)
        sgs = (sg0, sg1)
        sws = (sw0, sw1)
        pltpu.sync_copy(w_hbm.at[pl.ds(tbase * K, tpw * K)],
                        wv.at[pl.ds(0, tpw * K)])
        gathers = [None, None]
        writes = [None, None]
        pltpu.sync_copy(inv2_hbm.at[pl.ds(tbase, ct)], ix0.at[pl.ds(0, ct)])
        pltpu.sync_copy(inv2_hbm.at[pl.ds(T + tbase, ct)],
                        ix0.at[pl.ds(ct, ct)])
        gathers[0] = pltpu.async_copy(ysw_hbm.at[ix0], buf0, sg0)
        for c in range(nch):
            s = c % 2
            gathers[s].wait()
            if c + 1 < nch:
                s1 = (c + 1) % 2
                off1 = tbase + (c + 1) * ct
                pltpu.sync_copy(inv2_hbm.at[pl.ds(off1, ct)],
                                ixs[s1].at[pl.ds(0, ct)])
                pltpu.sync_copy(inv2_hbm.at[pl.ds(T + off1, ct)],
                                ixs[s1].at[pl.ds(ct, ct)])
                gathers[s1] = pltpu.async_copy(
                    ysw_hbm.at[ixs[s1]], bufs[s1], sgs[s1])
            if writes[s] is not None:
                writes[s].wait()
            buf = bufs[s]
            ob = obs[s]
            wbase = c * ct

            @plsc.parallel_loop(0, ct, step=1)
            def body(j, buf=buf, ob=ob, wbase=wbase):
                wvec = wv[pl.ds(K * (wbase + j), 16)]
                w0 = wvec[0]
                w1 = wvec[1]
                for u in range(D // 16):
                    v = u * 16
                    ob[j, pl.ds(v, 16)] = (
                        w0 * buf[j, pl.ds(v, 16)]
                        + w1 * buf[ct + j, pl.ds(v, 16)])
            writes[s] = pltpu.async_copy(
                ob, y_hbm.at[pl.ds(tbase + c * ct, ct)], sws[s])
        writes[0].wait()
        writes[1].wait()

    return k(ysw, inv2, weights.reshape(-1))


def _gmm_body(blk_r, eid_r, lo_r, hi_r, first_r,
              wait_r, slot_r, pref_r, prefe_r, prefs_r,
              xs_r, w1_hbm, w2_hbm, out_r,
              w1b, w2b, sem1, sem2):
    i = pl.program_id(0)
    hd = D // 2

    def _w_start(e, s):
        pltpu.make_async_copy(w1_hbm.at[e, pl.ds(0, hd)],
                              w1b.at[s, pl.ds(0, hd)], sem1.at[s]).start()
        pltpu.make_async_copy(w1_hbm.at[e, pl.ds(hd, hd)],
                              w1b.at[s, pl.ds(hd, hd)], sem1.at[s]).start()
        pltpu.make_async_copy(w2_hbm.at[e, pl.ds(0, hd)],
                              w2b.at[s, pl.ds(0, hd)], sem2.at[s]).start()
        pltpu.make_async_copy(w2_hbm.at[e, pl.ds(hd, hd)],
                              w2b.at[s, pl.ds(hd, hd)], sem2.at[s]).start()

    @pl.when(i == 0)
    def _():
        _w_start(eid_r[0], 0)

    @pl.when(pref_r[i] != 0)
    def _():
        _w_start(prefe_r[i], prefs_r[i])

    @pl.when(wait_r[i] != 0)
    def _():
        s = slot_r[i]
        pltpu.make_async_copy(w1_hbm.at[0, pl.ds(0, hd)],
                              w1b.at[s, pl.ds(0, hd)], sem1.at[s]).wait()
        pltpu.make_async_copy(w1_hbm.at[0, pl.ds(hd, hd)],
                              w1b.at[s, pl.ds(hd, hd)], sem1.at[s]).wait()
        pltpu.make_async_copy(w2_hbm.at[0, pl.ds(0, hd)],
                              w2b.at[s, pl.ds(0, hd)], sem2.at[s]).wait()
        pltpu.make_async_copy(w2_hbm.at[0, pl.ds(hd, hd)],
                              w2b.at[s, pl.ds(hd, hd)], sem2.at[s]).wait()

    cs = slot_r[i]
    lo = lo_r[i]
    hi = hi_r[i]
    rows = lax.broadcasted_iota(jnp.int32, (BT, 1), 0)
    mask = (rows >= lo) & (rows < hi)
    xb = jnp.where(mask, xs_r[...], 0.0)
    h = jnp.dot(xb, w1b[cs], preferred_element_type=jnp.float32)
    yv = h[:, :H]
    g = h[:, H:]
    act = yv * (g * jax.nn.sigmoid(g))
    o = jnp.dot(act, w2b[cs], preferred_element_type=jnp.float32)

    @pl.when(first_r[i] != 0)
    def _():
        out_r[...] = o

    @pl.when(first_r[i] == 0)
    def _():
        out_r[...] = out_r[...] + o


def _grouped_mlp(xs, W1, W2, blk_t, eid_t, lo_t, hi_t, first_t,
                 wait_t, slot_t, pref_t, prefe_t, prefs_t):
    grid_spec = pltpu.PrefetchScalarGridSpec(
        num_scalar_prefetch=10,
        grid=(MAXT,),
        in_specs=[
            pl.BlockSpec((BT, D), lambda i, *s: (s[0][i], 0)),
            pl.BlockSpec(memory_space=pl.ANY),
            pl.BlockSpec(memory_space=pl.ANY),
        ],
        out_specs=pl.BlockSpec((BT, D), lambda i, *s: (s[0][i], 0)),
        scratch_shapes=[
            pltpu.VMEM((2, D, 2 * H), jnp.float32),
            pltpu.VMEM((2, H, D), jnp.float32),
            pltpu.SemaphoreType.DMA((2,)),
            pltpu.SemaphoreType.DMA((2,)),
        ],
    )
    return pl.pallas_call(
        _gmm_body,
        grid_spec=grid_spec,
        out_shape=jax.ShapeDtypeStruct((N, D), jnp.float32),
    )(blk_t, eid_t, lo_t, hi_t, first_t,
      wait_t, slot_t, pref_t, prefe_t, prefs_t,
      xs, W1, W2)


def kernel(x, weights, indices, W1, W2):
    (inv2, blk_t, eid_t, lo_t, hi_t, first_t,
     wait_t, slot_t, pref_t, prefe_t, prefs_t) = _routing_metadata(
        weights, indices)
    xs = _sc_dispatch(x, inv2)
    ys = _grouped_mlp(xs, W1, W2, blk_t, eid_t, lo_t, hi_t, first_t,
                      wait_t, slot_t, pref_t, prefe_t, prefs_t)
    return _sc_pair_add(ys, inv2, weights)
